# baseline (device time: 23497 ns/iter reference)
import jax
import jax.numpy as jnp
from jax import lax
from jax.experimental import pallas as pl
from jax.experimental.pallas import tpu as pltpu

N_CHUNKS = 16


def kernel(x):
    _, m, n = x.shape
    half = n // 2
    rows = m // 2
    rc = rows // N_CHUNKS

    def body(x_ref, out_ref, send_stage, add_stage, x_recv, part, y_recv,
             lc_send_sems, lc_add_sems, st_my_sems, st_ot_sems,
             x_send_sems, x_recv_sems, y_send_sems, y_recv_sems):
        my_x = lax.axis_index("x")
        my_y = lax.axis_index("y")
        other_x = 1 - my_x
        other_y = 1 - my_y
        x_peer = (other_x, my_y)
        y_peer = (my_x, other_y)

        row0 = my_y * rows
        orow0 = other_y * rows

        barrier_sem = pltpu.get_barrier_semaphore()
        for dev in (x_peer, y_peer):
            pl.semaphore_signal(
                barrier_sem, inc=1,
                device_id=dev, device_id_type=pl.DeviceIdType.MESH,
            )
        pl.semaphore_wait(barrier_sem, 2)

        lc_send = []
        lc_add = []
        for k in range(N_CHUNKS):
            cs = pltpu.make_async_copy(
                x_ref.at[0, pl.ds(row0 + k * rc, rc),
                         pl.ds(other_x * half, half)],
                send_stage.at[k],
                lc_send_sems.at[k],
            )
            cs.start()
            lc_send.append(cs)
            ca = pltpu.make_async_copy(
                x_ref.at[0, pl.ds(row0 + k * rc, rc),
                         pl.ds(my_x * half, half)],
                add_stage.at[k],
                lc_add_sems.at[k],
            )
            ca.start()
            lc_add.append(ca)

        x_rdmas = []
        for k in range(N_CHUNKS):
            lc_send[k].wait()
            r = pltpu.make_async_remote_copy(
                src_ref=send_stage.at[k],
                dst_ref=x_recv.at[k],
                send_sem=x_send_sems.at[k],
                recv_sem=x_recv_sems.at[k],
                device_id=x_peer,
                device_id_type=pl.DeviceIdType.MESH,
            )
            r.start()
            x_rdmas.append(r)

        y_rdmas = []
        st_my = []
        for k in range(N_CHUNKS):
            x_rdmas[k].wait_recv()
            lc_add[k].wait()
            part[k] = add_stage[k] + x_recv[k]
            ry = pltpu.make_async_remote_copy(
                src_ref=part.at[k],
                dst_ref=y_recv.at[k],
                send_sem=y_send_sems.at[k],
                recv_sem=y_recv_sems.at[k],
                device_id=y_peer,
                device_id_type=pl.DeviceIdType.MESH,
            )
            ry.start()
            y_rdmas.append(ry)
            sm = pltpu.make_async_copy(
                part.at[k],
                out_ref.at[pl.ds(row0 + k * rc, rc), :],
                st_my_sems.at[k],
            )
            sm.start()
            st_my.append(sm)

        st_ot = []
        for k in range(N_CHUNKS):
            y_rdmas[k].wait_recv()
            so = pltpu.make_async_copy(
                y_recv.at[k],
                out_ref.at[pl.ds(orow0 + k * rc, rc), :],
                st_ot_sems.at[k],
            )
            so.start()
            st_ot.append(so)

        for k in range(N_CHUNKS):
            st_my[k].wait()
            st_ot[k].wait()
            x_rdmas[k].wait_send()
            y_rdmas[k].wait_send()

    return pl.pallas_call(
        body,
        out_shape=jax.ShapeDtypeStruct((m, half), jnp.float32),
        in_specs=[pl.BlockSpec(memory_space=pl.ANY)],
        out_specs=pl.BlockSpec(memory_space=pl.ANY),
        scratch_shapes=[
            pltpu.VMEM((N_CHUNKS, rc, half), jnp.float32),
            pltpu.VMEM((N_CHUNKS, rc, half), jnp.float32),
            pltpu.VMEM((N_CHUNKS, rc, half), jnp.float32),
            pltpu.VMEM((N_CHUNKS, rc, half), jnp.float32),
            pltpu.VMEM((N_CHUNKS, rc, half), jnp.float32),
            pltpu.SemaphoreType.DMA((N_CHUNKS,)),
            pltpu.SemaphoreType.DMA((N_CHUNKS,)),
            pltpu.SemaphoreType.DMA((N_CHUNKS,)),
            pltpu.SemaphoreType.DMA((N_CHUNKS,)),
            pltpu.SemaphoreType.DMA((N_CHUNKS,)),
            pltpu.SemaphoreType.DMA((N_CHUNKS,)),
            pltpu.SemaphoreType.DMA((N_CHUNKS,)),
            pltpu.SemaphoreType.DMA((N_CHUNKS,)),
        ],
        compiler_params=pltpu.CompilerParams(collective_id=0),
    )(x)


# device time: 23097 ns/iter; 1.0173x vs baseline; 1.0173x over previous
import jax
import jax.numpy as jnp
from jax import lax
from jax.experimental import pallas as pl
from jax.experimental.pallas import tpu as pltpu

N_CHUNKS = 16


def kernel(x):
    _, m, n = x.shape
    half = n // 2
    rows = m // 2
    rc = rows // N_CHUNKS

    def body(x_ref, out_ref, send_stage, add_stage, x_recv,
             lc_send_sems, lc_add_sems,
             x_send_sems, x_recv_sems, y_send_sems, y_recv_sems):
        my_x = lax.axis_index("x")
        my_y = lax.axis_index("y")
        other_x = 1 - my_x
        other_y = 1 - my_y
        x_peer = (other_x, my_y)
        y_peer = (my_x, other_y)

        row0 = my_y * rows

        barrier_sem = pltpu.get_barrier_semaphore()
        for dev in (x_peer, y_peer):
            pl.semaphore_signal(
                barrier_sem, inc=1,
                device_id=dev, device_id_type=pl.DeviceIdType.MESH,
            )
        pl.semaphore_wait(barrier_sem, 2)

        lc_send = []
        lc_add = []
        for k in range(N_CHUNKS):
            cs = pltpu.make_async_copy(
                x_ref.at[0, pl.ds(row0 + k * rc, rc),
                         pl.ds(other_x * half, half)],
                send_stage.at[k],
                lc_send_sems.at[k],
            )
            cs.start()
            lc_send.append(cs)
            ca = pltpu.make_async_copy(
                x_ref.at[0, pl.ds(row0 + k * rc, rc),
                         pl.ds(my_x * half, half)],
                add_stage.at[k],
                lc_add_sems.at[k],
            )
            ca.start()
            lc_add.append(ca)

        x_rdmas = []
        for k in range(N_CHUNKS):
            lc_send[k].wait()
            r = pltpu.make_async_remote_copy(
                src_ref=send_stage.at[k],
                dst_ref=x_recv.at[k],
                send_sem=x_send_sems.at[k],
                recv_sem=x_recv_sems.at[k],
                device_id=x_peer,
                device_id_type=pl.DeviceIdType.MESH,
            )
            r.start()
            x_rdmas.append(r)

        y_rdmas = []
        for k in range(N_CHUNKS):
            x_rdmas[k].wait_recv()
            lc_add[k].wait()
            out_ref[pl.ds(row0 + k * rc, rc), :] = add_stage[k] + x_recv[k]
            ry = pltpu.make_async_remote_copy(
                src_ref=out_ref.at[pl.ds(row0 + k * rc, rc), :],
                dst_ref=out_ref.at[pl.ds(row0 + k * rc, rc), :],
                send_sem=y_send_sems.at[k],
                recv_sem=y_recv_sems.at[k],
                device_id=y_peer,
                device_id_type=pl.DeviceIdType.MESH,
            )
            ry.start()
            y_rdmas.append(ry)

        for k in range(N_CHUNKS):
            y_rdmas[k].wait_recv()
        for k in range(N_CHUNKS):
            x_rdmas[k].wait_send()
            y_rdmas[k].wait_send()

    return pl.pallas_call(
        body,
        out_shape=jax.ShapeDtypeStruct((m, half), jnp.float32),
        in_specs=[pl.BlockSpec(memory_space=pl.ANY)],
        out_specs=pl.BlockSpec(memory_space=pltpu.VMEM),
        scratch_shapes=[
            pltpu.VMEM((N_CHUNKS, rc, half), jnp.float32),
            pltpu.VMEM((N_CHUNKS, rc, half), jnp.float32),
            pltpu.VMEM((N_CHUNKS, rc, half), jnp.float32),
            pltpu.SemaphoreType.DMA((N_CHUNKS,)),
            pltpu.SemaphoreType.DMA((N_CHUNKS,)),
            pltpu.SemaphoreType.DMA((N_CHUNKS,)),
            pltpu.SemaphoreType.DMA((N_CHUNKS,)),
            pltpu.SemaphoreType.DMA((N_CHUNKS,)),
            pltpu.SemaphoreType.DMA((N_CHUNKS,)),
        ],
        compiler_params=pltpu.CompilerParams(collective_id=0),
    )(x)


# device time: 22127 ns/iter; 1.0619x vs baseline; 1.0438x over previous
import jax
import jax.numpy as jnp
from jax import lax
from jax.experimental import pallas as pl
from jax.experimental.pallas import tpu as pltpu

N_CHUNKS = 16


def kernel(x):
    _, m, n = x.shape
    half = n // 2
    rows = m // 2
    rc = rows // N_CHUNKS

    def body(x_ref, out_ref, x_recv,
             x_send_sems, x_recv_sems, y_send_sems, y_recv_sems):
        my_x = lax.axis_index("x")
        my_y = lax.axis_index("y")
        other_x = 1 - my_x
        other_y = 1 - my_y
        x_peer = (other_x, my_y)
        y_peer = (my_x, other_y)

        row0 = my_y * rows

        barrier_sem = pltpu.get_barrier_semaphore()
        for dev in (x_peer, y_peer):
            pl.semaphore_signal(
                barrier_sem, inc=1,
                device_id=dev, device_id_type=pl.DeviceIdType.MESH,
            )
        pl.semaphore_wait(barrier_sem, 2)

        x_rdmas = []
        for k in range(N_CHUNKS):
            r = pltpu.make_async_remote_copy(
                src_ref=x_ref.at[0, pl.ds(row0 + k * rc, rc),
                                 pl.ds(other_x * half, half)],
                dst_ref=x_recv.at[k],
                send_sem=x_send_sems.at[k],
                recv_sem=x_recv_sems.at[k],
                device_id=x_peer,
                device_id_type=pl.DeviceIdType.MESH,
            )
            r.start()
            x_rdmas.append(r)

        y_rdmas = []
        for k in range(N_CHUNKS):
            x_rdmas[k].wait_recv()
            out_ref[pl.ds(row0 + k * rc, rc), :] = (
                x_ref[0, pl.ds(row0 + k * rc, rc),
                      pl.ds(my_x * half, half)]
                + x_recv[k]
            )
            ry = pltpu.make_async_remote_copy(
                src_ref=out_ref.at[pl.ds(row0 + k * rc, rc), :],
                dst_ref=out_ref.at[pl.ds(row0 + k * rc, rc), :],
                send_sem=y_send_sems.at[k],
                recv_sem=y_recv_sems.at[k],
                device_id=y_peer,
                device_id_type=pl.DeviceIdType.MESH,
            )
            ry.start()
            y_rdmas.append(ry)

        for k in range(N_CHUNKS):
            y_rdmas[k].wait_recv()
        for k in range(N_CHUNKS):
            x_rdmas[k].wait_send()
            y_rdmas[k].wait_send()

    return pl.pallas_call(
        body,
        out_shape=jax.ShapeDtypeStruct((m, half), jnp.float32),
        in_specs=[pl.BlockSpec(memory_space=pltpu.VMEM)],
        out_specs=pl.BlockSpec(memory_space=pltpu.VMEM),
        scratch_shapes=[
            pltpu.VMEM((N_CHUNKS, rc, half), jnp.float32),
            pltpu.SemaphoreType.DMA((N_CHUNKS,)),
            pltpu.SemaphoreType.DMA((N_CHUNKS,)),
            pltpu.SemaphoreType.DMA((N_CHUNKS,)),
            pltpu.SemaphoreType.DMA((N_CHUNKS,)),
        ],
        compiler_params=pltpu.CompilerParams(collective_id=0),
    )(x)


# device time: 20682 ns/iter; 1.1361x vs baseline; 1.0699x over previous
import jax
import jax.numpy as jnp
from jax import lax
from jax.experimental import pallas as pl
from jax.experimental.pallas import tpu as pltpu

N_CHUNKS = 8


def kernel(x):
    _, m, n = x.shape
    half = n // 2
    rows = m // 2
    rc = rows // N_CHUNKS

    my_y_outer = lax.axis_index("y")
    x = lax.dynamic_slice(x, (0, my_y_outer * rows, 0), (1, rows, n))

    def body(x_ref, out_ref, x_recv,
             x_send_sems, x_recv_sems, y_send_sems, y_recv_sems):
        my_x = lax.axis_index("x")
        my_y = lax.axis_index("y")
        other_x = 1 - my_x
        other_y = 1 - my_y
        x_peer = (other_x, my_y)
        y_peer = (my_x, other_y)

        row0 = my_y * rows

        barrier_sem = pltpu.get_barrier_semaphore()
        for dev in (x_peer, y_peer):
            pl.semaphore_signal(
                barrier_sem, inc=1,
                device_id=dev, device_id_type=pl.DeviceIdType.MESH,
            )
        pl.semaphore_wait(barrier_sem, 2)

        x_rdmas = []
        for k in range(N_CHUNKS):
            r = pltpu.make_async_remote_copy(
                src_ref=x_ref.at[0, pl.ds(k * rc, rc),
                                 pl.ds(other_x * half, half)],
                dst_ref=x_recv.at[k],
                send_sem=x_send_sems.at[k],
                recv_sem=x_recv_sems.at[k],
                device_id=x_peer,
                device_id_type=pl.DeviceIdType.MESH,
            )
            r.start()
            x_rdmas.append(r)

        y_rdmas = []
        for k in range(N_CHUNKS):
            x_rdmas[k].wait_recv()
            out_ref[pl.ds(row0 + k * rc, rc), :] = (
                x_ref[0, pl.ds(k * rc, rc), pl.ds(my_x * half, half)]
                + x_recv[k]
            )
            ry = pltpu.make_async_remote_copy(
                src_ref=out_ref.at[pl.ds(row0 + k * rc, rc), :],
                dst_ref=out_ref.at[pl.ds(row0 + k * rc, rc), :],
                send_sem=y_send_sems.at[k],
                recv_sem=y_recv_sems.at[k],
                device_id=y_peer,
                device_id_type=pl.DeviceIdType.MESH,
            )
            ry.start()
            y_rdmas.append(ry)

        for k in range(N_CHUNKS):
            y_rdmas[k].wait_recv()
        for k in range(N_CHUNKS):
            x_rdmas[k].wait_send()
            y_rdmas[k].wait_send()

    return pl.pallas_call(
        body,
        out_shape=jax.ShapeDtypeStruct((m, half), jnp.float32),
        in_specs=[pl.BlockSpec(memory_space=pltpu.VMEM)],
        out_specs=pl.BlockSpec(memory_space=pltpu.VMEM),
        scratch_shapes=[
            pltpu.VMEM((N_CHUNKS, rc, half), jnp.float32),
            pltpu.SemaphoreType.DMA((N_CHUNKS,)),
            pltpu.SemaphoreType.DMA((N_CHUNKS,)),
            pltpu.SemaphoreType.DMA((N_CHUNKS,)),
            pltpu.SemaphoreType.DMA((N_CHUNKS,)),
        ],
        compiler_params=pltpu.CompilerParams(collective_id=0),
    )(x)


# device time: 20273 ns/iter; 1.1590x vs baseline; 1.0202x over previous
import jax
import jax.numpy as jnp
from jax import lax
from jax.experimental import pallas as pl
from jax.experimental.pallas import tpu as pltpu

N_CHUNKS = 16


def kernel(x):
    _, m, n = x.shape
    half = n // 2
    rows = m // 2
    rc = rows // N_CHUNKS

    my_y_outer = lax.axis_index("y")
    x = lax.dynamic_slice(x, (0, my_y_outer * rows, 0), (1, rows, n))

    def body(x_ref, out_ref, x_recv,
             x_send_sems, x_recv_sems, y_send_sems, y_recv_sems):
        my_x = lax.axis_index("x")
        my_y = lax.axis_index("y")
        other_x = 1 - my_x
        other_y = 1 - my_y
        x_peer = (other_x, my_y)
        y_peer = (my_x, other_y)

        row0 = my_y * rows

        barrier_sem = pltpu.get_barrier_semaphore()
        for dev in (x_peer, y_peer):
            pl.semaphore_signal(
                barrier_sem, inc=1,
                device_id=dev, device_id_type=pl.DeviceIdType.MESH,
            )
        pl.semaphore_wait(barrier_sem, 2)

        x_rdmas = []
        for k in range(N_CHUNKS):
            r = pltpu.make_async_remote_copy(
                src_ref=x_ref.at[0, pl.ds(k * rc, rc),
                                 pl.ds(other_x * half, half)],
                dst_ref=x_recv.at[k],
                send_sem=x_send_sems.at[k],
                recv_sem=x_recv_sems.at[k],
                device_id=x_peer,
                device_id_type=pl.DeviceIdType.MESH,
            )
            r.start()
            x_rdmas.append(r)

        y_rdmas = []
        for k in range(N_CHUNKS):
            x_rdmas[k].wait_recv()
            out_ref[pl.ds(row0 + k * rc, rc), :] = (
                x_ref[0, pl.ds(k * rc, rc), pl.ds(my_x * half, half)]
                + x_recv[k]
            )
            ry = pltpu.make_async_remote_copy(
                src_ref=out_ref.at[pl.ds(row0 + k * rc, rc), :],
                dst_ref=out_ref.at[pl.ds(row0 + k * rc, rc), :],
                send_sem=y_send_sems.at[k],
                recv_sem=y_recv_sems.at[k],
                device_id=y_peer,
                device_id_type=pl.DeviceIdType.MESH,
            )
            ry.start()
            y_rdmas.append(ry)

        for k in range(N_CHUNKS):
            y_rdmas[k].wait_recv()
        for k in range(N_CHUNKS):
            x_rdmas[k].wait_send()
            y_rdmas[k].wait_send()

    return pl.pallas_call(
        body,
        out_shape=jax.ShapeDtypeStruct((m, half), jnp.float32),
        in_specs=[pl.BlockSpec(memory_space=pltpu.VMEM)],
        out_specs=pl.BlockSpec(memory_space=pltpu.VMEM),
        scratch_shapes=[
            pltpu.VMEM((N_CHUNKS, rc, half), jnp.float32),
            pltpu.SemaphoreType.DMA((N_CHUNKS,)),
            pltpu.SemaphoreType.DMA((N_CHUNKS,)),
            pltpu.SemaphoreType.DMA((N_CHUNKS,)),
            pltpu.SemaphoreType.DMA((N_CHUNKS,)),
        ],
        compiler_params=pltpu.CompilerParams(collective_id=0),
    )(x)
